# baseline (speedup 1.0000x reference)
import functools

import jax
import jax.numpy as jnp
from jax import lax
from jax.experimental import pallas as pl
from jax.experimental.pallas import tpu as pltpu
from jax.experimental.pallas import tpu_sc as plsc

_NUM_CODES = 1024
_EMBED_DIM = 64
_COMMIT = 0.25
_EPS = 1e-10
_BLOCK_T = 2048
_NUM_WORKERS = 32


def _vq_body(z_ref, e_ref, idx_ref, loss_ref, perp_ref,
             counts, acc, esq_s, e2_s):
    i = pl.program_id(0)
    k = pl.num_programs(0)
    t = z_ref.shape[0]
    n = t * k

    @pl.when(i == 0)
    def _init():
        counts[...] = jnp.zeros_like(counts)
        acc[...] = jnp.zeros_like(acc)
        e = e_ref[...]
        e2_s[...] = e * -2.0
        esq_s[...] = jnp.sum(e * e, axis=1)[None, :]

    z = z_ref[...]
    zsq = jnp.sum(z * z, axis=1, keepdims=True)
    prod2 = jax.lax.dot_general(
        z, e2_s[...], (((1,), (1,)), ((), ())),
        preferred_element_type=jnp.float32)
    dist = zsq + prod2 + esq_s[...]
    idx = jnp.argmin(dist, axis=1).astype(jnp.int32)
    m = jnp.min(dist, axis=1, keepdims=True)
    iota = jax.lax.broadcasted_iota(jnp.int32, dist.shape, 1)
    onehot = (iota == idx[:, None]).astype(jnp.float32)
    idx_ref[...] = idx[:, None]
    counts[...] += jax.lax.dot_general(
        jnp.ones((1, t), jnp.float32), onehot, (((1,), (0,)), ((), ())),
        preferred_element_type=jnp.float32)
    acc[...] += jnp.sum(m, axis=(0, 1), keepdims=True)

    @pl.when(i == k - 1)
    def _fin():
        avg = counts[...] / n
        ent = jnp.sum(avg * jnp.log(avg + _EPS), axis=1, keepdims=True)
        perp_ref[...] = jnp.exp(-ent)
        loss_ref[...] = _COMMIT * acc[...] / (n * _EMBED_DIM)


def _pick_chunk(per_w):
    for c in range(min(per_w, 128), 7, -1):
        if per_w % c == 0 and c % 8 == 0:
            return c
    return None


_PAD_DIM = 128


def _make_sc_gather(n, per_w, n_chunks, chunk):
    mesh = plsc.VectorSubcoreMesh(core_axis_name="c", subcore_axis_name="s")

    @functools.partial(
        pl.kernel,
        mesh=mesh,
        out_type=jax.ShapeDtypeStruct((n, _PAD_DIM), jnp.float32),
        scratch_types=[
            pltpu.VMEM((n_chunks, chunk), jnp.int32),
            pltpu.VMEM((per_w, _PAD_DIM), jnp.float32),
            pltpu.SemaphoreType.DMA,
            pltpu.SemaphoreType.DMA,
        ],
    )
    def sc_gather(idx_hbm, table_hbm, out_hbm, idx_v, rows_v, gsem, wsem):
        wid = lax.axis_index("s") * 2 + lax.axis_index("c")
        base = wid * per_w
        pltpu.sync_copy(idx_hbm.at[wid], idx_v)
        gathers = []
        for j in range(n_chunks):
            gathers.append(pltpu.async_copy(
                table_hbm.at[idx_v.at[j]],
                rows_v.at[pl.ds(j * chunk, chunk)],
                gsem))
        writes = []
        for j in range(n_chunks):
            gathers[j].wait()
            writes.append(pltpu.async_copy(
                rows_v.at[pl.ds(j * chunk, chunk)],
                out_hbm.at[pl.ds(base + j * chunk, chunk)],
                wsem))
        for w in writes:
            w.wait()

    return sc_gather


def _tc_part(zh, embedding, t):
    nh = zh.shape[0]
    k = nh // t
    return pl.pallas_call(
        _vq_body,
        grid=(k,),
        in_specs=[
            pl.BlockSpec((t, _EMBED_DIM), lambda i: (i, 0)),
            pl.BlockSpec((_NUM_CODES, _EMBED_DIM), lambda i: (0, 0)),
        ],
        out_specs=[
            pl.BlockSpec((t, 1), lambda i: (i, 0)),
            pl.BlockSpec((1, 1), lambda i: (0, 0)),
            pl.BlockSpec((1, 1), lambda i: (0, 0)),
        ],
        out_shape=[
            jax.ShapeDtypeStruct((nh, 1), jnp.int32),
            jax.ShapeDtypeStruct((1, 1), jnp.float32),
            jax.ShapeDtypeStruct((1, 1), jnp.float32),
        ],
        scratch_shapes=[
            pltpu.VMEM((1, _NUM_CODES), jnp.float32),
            pltpu.VMEM((1, 1), jnp.float32),
            pltpu.VMEM((1, _NUM_CODES), jnp.float32),
            pltpu.VMEM((_NUM_CODES, _EMBED_DIM), jnp.float32),
        ],
    )(zh, embedding)


def _sc_gather_part(idx_h, table, nh):
    per_w = nh // _NUM_WORKERS
    chunk = _pick_chunk(per_w)
    n_chunks = per_w // chunk
    idx_g = idx_h.reshape(_NUM_WORKERS, n_chunks, chunk)
    return _make_sc_gather(nh, per_w, n_chunks, chunk)(idx_g, table)


def kernel(z, embedding):
    shape = z.shape
    zf = z.reshape(-1, _EMBED_DIM)
    n = zf.shape[0]
    t = _BLOCK_T if n % _BLOCK_T == 0 else n
    idx, loss, perp = _tc_part(zf, embedding, t)
    sc_ok = n % _NUM_WORKERS == 0 and _pick_chunk(n // _NUM_WORKERS) is not None
    if sc_ok:
        table = jnp.pad(embedding, ((0, 0), (0, _PAD_DIM - _EMBED_DIM)))
        zq = _sc_gather_part(idx, table, n)[:, :_EMBED_DIM]
    else:
        zq = jax.nn.one_hot(idx[:, 0], _NUM_CODES, dtype=jnp.float32) @ embedding
    return (
        zq.reshape(shape),
        idx[:, 0].reshape(shape[:-1]),
        loss[0, 0],
        perp[0, 0],
    )


# speedup vs baseline: 1.0520x; 1.0520x over previous
import functools

import jax
import jax.numpy as jnp
from jax import lax
from jax.experimental import pallas as pl
from jax.experimental.pallas import tpu as pltpu
from jax.experimental.pallas import tpu_sc as plsc

_NUM_CODES = 1024
_EMBED_DIM = 64
_COMMIT = 0.25
_EPS = 1e-10
_BLOCK_T = 3072
_NUM_WORKERS = 32


def _vq_body(z_ref, e_ref, idx_ref, loss_ref, perp_ref,
             counts, acc, esq_s, e2_s):
    i = pl.program_id(0)
    k = pl.num_programs(0)
    t = z_ref.shape[0]
    n = t * k

    @pl.when(i == 0)
    def _init():
        counts[...] = jnp.zeros_like(counts)
        acc[...] = jnp.zeros_like(acc)
        e = e_ref[...]
        e2_s[...] = e * -2.0
        esq_s[...] = jnp.sum(e * e, axis=1)[None, :]

    z = z_ref[...]
    zsq = jnp.sum(z * z, axis=1, keepdims=True)
    prod2 = jax.lax.dot_general(
        z, e2_s[...], (((1,), (1,)), ((), ())),
        preferred_element_type=jnp.float32)
    dist = zsq + prod2 + esq_s[...]
    idx = jnp.argmin(dist, axis=1).astype(jnp.int32)
    m = jnp.min(dist, axis=1, keepdims=True)
    iota = jax.lax.broadcasted_iota(jnp.int32, dist.shape, 1)
    onehot = (iota == idx[:, None]).astype(jnp.float32)
    idx_ref[...] = idx[:, None]
    counts[...] += jax.lax.dot_general(
        jnp.ones((1, t), jnp.float32), onehot, (((1,), (0,)), ((), ())),
        preferred_element_type=jnp.float32)
    acc[...] += jnp.sum(m, axis=(0, 1), keepdims=True)

    @pl.when(i == k - 1)
    def _fin():
        avg = counts[...] / n
        ent = jnp.sum(avg * jnp.log(avg + _EPS), axis=1, keepdims=True)
        perp_ref[...] = jnp.exp(-ent)
        loss_ref[...] = _COMMIT * acc[...] / (n * _EMBED_DIM)


def _pick_chunk(per_w):
    for c in range(min(per_w, 128), 7, -1):
        if per_w % c == 0 and c % 8 == 0:
            return c
    return None


_PAD_DIM = 128


def _make_sc_gather(n, per_w, n_chunks, chunk):
    mesh = plsc.VectorSubcoreMesh(core_axis_name="c", subcore_axis_name="s")

    @functools.partial(
        pl.kernel,
        mesh=mesh,
        out_type=jax.ShapeDtypeStruct((n, _PAD_DIM), jnp.float32),
        scratch_types=[
            pltpu.VMEM((n_chunks, chunk), jnp.int32),
            pltpu.VMEM((per_w, _PAD_DIM), jnp.float32),
            pltpu.SemaphoreType.DMA,
        ],
    )
    def sc_gather(idx_hbm, table_hbm, out_hbm, idx_v, rows_v, sem):
        wid = lax.axis_index("s") * 2 + lax.axis_index("c")
        base = wid * per_w
        pltpu.sync_copy(idx_hbm.at[wid], idx_v)
        copies = []
        for j in range(n_chunks):
            copies.append(pltpu.async_copy(
                table_hbm.at[idx_v.at[j]],
                rows_v.at[pl.ds(j * chunk, chunk)],
                sem))
        for c in copies:
            c.wait()
        pltpu.sync_copy(rows_v, out_hbm.at[pl.ds(base, per_w)])

    return sc_gather


def _tc_part(zh, embedding, t):
    nh = zh.shape[0]
    k = nh // t
    return pl.pallas_call(
        _vq_body,
        grid=(k,),
        in_specs=[
            pl.BlockSpec((t, _EMBED_DIM), lambda i: (i, 0)),
            pl.BlockSpec((_NUM_CODES, _EMBED_DIM), lambda i: (0, 0)),
        ],
        out_specs=[
            pl.BlockSpec((t, 1), lambda i: (i, 0)),
            pl.BlockSpec((1, 1), lambda i: (0, 0)),
            pl.BlockSpec((1, 1), lambda i: (0, 0)),
        ],
        out_shape=[
            jax.ShapeDtypeStruct((nh, 1), jnp.int32),
            jax.ShapeDtypeStruct((1, 1), jnp.float32),
            jax.ShapeDtypeStruct((1, 1), jnp.float32),
        ],
        scratch_shapes=[
            pltpu.VMEM((1, _NUM_CODES), jnp.float32),
            pltpu.VMEM((1, 1), jnp.float32),
            pltpu.VMEM((1, _NUM_CODES), jnp.float32),
            pltpu.VMEM((_NUM_CODES, _EMBED_DIM), jnp.float32),
        ],
    )(zh, embedding)


def _sc_gather_part(idx_h, table, nh):
    per_w = nh // _NUM_WORKERS
    chunk = _pick_chunk(per_w)
    n_chunks = per_w // chunk
    idx_g = idx_h.reshape(_NUM_WORKERS, n_chunks, chunk)
    return _make_sc_gather(nh, per_w, n_chunks, chunk)(idx_g, table)


def kernel(z, embedding):
    shape = z.shape
    zf = z.reshape(-1, _EMBED_DIM)
    n = zf.shape[0]
    t = _BLOCK_T if n % _BLOCK_T == 0 else n
    idx, loss, perp = _tc_part(zf, embedding, t)
    sc_ok = n % _NUM_WORKERS == 0 and _pick_chunk(n // _NUM_WORKERS) is not None
    if sc_ok:
        table = jnp.pad(embedding, ((0, 0), (0, _PAD_DIM - _EMBED_DIM)))
        zq = _sc_gather_part(idx, table, n)[:, :_EMBED_DIM]
    else:
        zq = jax.nn.one_hot(idx[:, 0], _NUM_CODES, dtype=jnp.float32) @ embedding
    return (
        zq.reshape(shape),
        idx[:, 0].reshape(shape[:-1]),
        loss[0, 0],
        perp[0, 0],
    )
